# Initial kernel scaffold; baseline (speedup 1.0000x reference)
import jax
import jax.numpy as jnp
from jax.experimental import pallas as pl


def kernel(x, edge_index, edge_attr, inW, inb, ing, inbt, Wq, Wk, Wv, We, be, Wo, bo, g1, b1, g2, b2, F1, bf1, F2, bf2, cW1, cb1, cW2, cb2):
    raise NotImplementedError("write your pallas kernel here")


# speedup vs baseline: 1.0152x; 1.0152x over previous
import functools

import jax
import jax.numpy as jnp
from jax import lax
from jax.experimental import pallas as pl
from jax.experimental.pallas import tpu as pltpu

N = 10000
E = 160000
D = 256
H = 8
DH = D // H
L = 5
ED = 4
SCALE = float(jnp.sqrt(jnp.float32(DH)))

NP = 10240


def _ln_rows(x, g, b):
    mu = jnp.mean(x, axis=-1, keepdims=True)
    var = jnp.mean((x - mu) ** 2, axis=-1, keepdims=True)
    return (x - mu) * jax.lax.rsqrt(var + 1e-5) * g + b


def _gelu(x):
    return 0.5 * x * (1.0 + lax.erf(x * 0.7071067811865476))


def _dotT(a, w):
    return lax.dot_general(a, w, (((1,), (1,)), ((), ())),
                           preferred_element_type=jnp.float32)


def _in_body(x_ref, w_ref, b_ref, g_ref, bt_ref, o_ref):
    h = _dotT(x_ref[...], w_ref[...]) + b_ref[...]
    o_ref[...] = _gelu(_ln_rows(h, g_ref[...], bt_ref[...]))


def _input_mlp(x, inW, inb, ing, inbt):
    B = 512
    return pl.pallas_call(
        _in_body,
        grid=(NP // B,),
        in_specs=[
            pl.BlockSpec((B, D), lambda i: (i, 0)),
            pl.BlockSpec((D, D), lambda i: (0, 0)),
            pl.BlockSpec((D,), lambda i: (0,)),
            pl.BlockSpec((D,), lambda i: (0,)),
            pl.BlockSpec((D,), lambda i: (0,)),
        ],
        out_specs=pl.BlockSpec((B, D), lambda i: (i, 0)),
        out_shape=jax.ShapeDtypeStruct((NP, D), jnp.float32),
    )(x, inW, inb, ing, inbt)


def _qkv_body(h_ref, g_ref, b_ref, wq_ref, wk_ref, wv_ref, q_ref, k_ref, v_ref):
    hn = _ln_rows(h_ref[...], g_ref[...], b_ref[...])
    q_ref[...] = _dotT(hn, wq_ref[...])
    k_ref[...] = _dotT(hn, wk_ref[...])
    v_ref[...] = _dotT(hn, wv_ref[...])


def _qkv(h, g1, b1, Wq, Wk, Wv):
    B = 512
    out = jax.ShapeDtypeStruct((NP, D), jnp.float32)
    return pl.pallas_call(
        _qkv_body,
        grid=(NP // B,),
        in_specs=[
            pl.BlockSpec((B, D), lambda i: (i, 0)),
            pl.BlockSpec((D,), lambda i: (0,)),
            pl.BlockSpec((D,), lambda i: (0,)),
            pl.BlockSpec((D, D), lambda i: (0, 0)),
            pl.BlockSpec((D, D), lambda i: (0, 0)),
            pl.BlockSpec((D, D), lambda i: (0, 0)),
        ],
        out_specs=[pl.BlockSpec((B, D), lambda i: (i, 0))] * 3,
        out_shape=[out, out, out],
    )(h, g1, b1, Wq, Wk, Wv)


def _post_body(num_ref, den_ref, h_ref, wo_ref, bo_ref, g2_ref, b2_ref,
               f1_ref, bf1_ref, f2_ref, bf2_ref, r_ref, o_ref):
    den_full = jnp.dot(den_ref[...], r_ref[...],
                       preferred_element_type=jnp.float32)
    agg = num_ref[...] / (den_full + 1e-16)
    out = _dotT(agg, wo_ref[...]) + bo_ref[...] + h_ref[...]
    out = _ln_rows(out, g2_ref[...], b2_ref[...])
    f = _gelu(_dotT(out, f1_ref[...]) + bf1_ref[...])
    f = _dotT(f, f2_ref[...]) + bf2_ref[...]
    o_ref[...] = out + f


def _post(num, den, h, Wo, bo, g2, b2, F1, bf1, F2, bf2, R):
    B = 256
    return pl.pallas_call(
        _post_body,
        grid=(NP // B,),
        in_specs=[
            pl.BlockSpec((B, D), lambda i: (i, 0)),
            pl.BlockSpec((B, 16), lambda i: (i, 0)),
            pl.BlockSpec((B, D), lambda i: (i, 0)),
            pl.BlockSpec((D, D), lambda i: (0, 0)),
            pl.BlockSpec((D,), lambda i: (0,)),
            pl.BlockSpec((D,), lambda i: (0,)),
            pl.BlockSpec((D,), lambda i: (0,)),
            pl.BlockSpec((4 * D, D), lambda i: (0, 0)),
            pl.BlockSpec((4 * D,), lambda i: (0,)),
            pl.BlockSpec((D, 4 * D), lambda i: (0, 0)),
            pl.BlockSpec((D,), lambda i: (0,)),
            pl.BlockSpec((16, D), lambda i: (0, 0)),
        ],
        out_specs=pl.BlockSpec((B, D), lambda i: (i, 0)),
        out_shape=jax.ShapeDtypeStruct((NP, D), jnp.float32),
    )(num, den, h, Wo, bo, g2, b2, F1, bf1, F2, bf2, R)


def _cls_body(h_ref, w1_ref, b1_ref, w2_ref, b2_ref, o_ref):
    z = _gelu(_dotT(h_ref[...], w1_ref[...]) + b1_ref[...])
    o_ref[...] = _dotT(z, w2_ref[...]) + b2_ref[...]


def _classifier(h, cW1, cb1, cW2, cb2):
    B = 512
    cW2p = jnp.zeros((128, D // 2), jnp.float32).at[:2].set(cW2)
    cb2p = jnp.zeros((128,), jnp.float32).at[:2].set(cb2)
    out = pl.pallas_call(
        _cls_body,
        grid=(NP // B,),
        in_specs=[
            pl.BlockSpec((B, D), lambda i: (i, 0)),
            pl.BlockSpec((D // 2, D), lambda i: (0, 0)),
            pl.BlockSpec((D // 2,), lambda i: (0,)),
            pl.BlockSpec((128, D // 2), lambda i: (0, 0)),
            pl.BlockSpec((128,), lambda i: (0,)),
        ],
        out_specs=pl.BlockSpec((B, 128), lambda i: (i, 0)),
        out_shape=jax.ShapeDtypeStruct((NP, 128), jnp.float32),
    )(h, cW1, cb1, cW2p, cb2p)
    return out[:N, :2]


def _eb_body(ea_ref, we_ref, be_ref, o_ref):
    w = we_ref[...].reshape(H, ED)
    b = be_ref[...].reshape(1, H)
    o_ref[...] = (_dotT(ea_ref[...], w) + b)[None]


def _edge_bias(ea, We, be):
    BE = 8000
    return pl.pallas_call(
        _eb_body,
        grid=(L, E // BE),
        in_specs=[
            pl.BlockSpec((BE, ED), lambda l, e: (e, 0)),
            pl.BlockSpec((1, H, ED), lambda l, e: (l, 0, 0)),
            pl.BlockSpec((1, 1, H), lambda l, e: (l, 0, 0)),
        ],
        out_specs=pl.BlockSpec((1, BE, H), lambda l, e: (l, e, 0)),
        out_shape=jax.ShapeDtypeStruct((L, E, H), jnp.float32),
    )(ea, We, be.reshape(L, 1, H))


def _edge_phase_xla(q, k, v, src, dst, eb):
    qi = q[:N].reshape(N, H, DH)[dst]
    kj = k[:N].reshape(N, H, DH)[src]
    vj = v[:N].reshape(N, H, DH)[src]
    score = jnp.sum(qi * kj, axis=-1) / SCALE + eb
    m = jax.ops.segment_max(score, dst, num_segments=N)
    m = jnp.where(jnp.isfinite(m), m, 0.0)
    ex = jnp.exp(score - m[dst])
    s = jax.ops.segment_sum(ex, dst, num_segments=N)
    num = jax.ops.segment_sum(ex[..., None] * vj, dst, num_segments=N)
    num = num.reshape(N, D)
    numP = jnp.zeros((NP, D), jnp.float32).at[:N].set(num)
    denP = jnp.zeros((NP, 16), jnp.float32).at[:N, :H].set(s)
    return numP, denP


def kernel(x, edge_index, edge_attr, inW, inb, ing, inbt, Wq, Wk, Wv, We, be,
           Wo, bo, g1, b1, g2, b2, F1, bf1, F2, bf2, cW1, cb1, cW2, cb2):
    src = edge_index[0]
    dst = edge_index[1]

    xP = jnp.zeros((NP, D), jnp.float32).at[:N].set(x)
    R = jnp.zeros((16, D), jnp.float32)
    R = R.at[jnp.arange(D) // DH, jnp.arange(D)].set(1.0)

    eb_all = _edge_bias(edge_attr, We, be)

    h = _input_mlp(xP, inW, inb, ing, inbt)
    for l in range(L):
        q, k, v = _qkv(h, g1[l], b1[l], Wq[l], Wk[l], Wv[l])
        num, den = _edge_phase_xla(q, k, v, src, dst, eb_all[l])
        h = _post(num, den, h, Wo[l], bo[l], g2[l], b2[l], F1[l], bf1[l],
                  F2[l], bf2[l], R)

    logits = _classifier(h, cW1, cb1, cW2, cb2)
    return logits, h[:N]


# speedup vs baseline: 7.9068x; 7.7888x over previous
import functools
import math

import jax
import jax.numpy as jnp
from jax import lax
from jax.experimental import pallas as pl
from jax.experimental.pallas import tpu as pltpu
from jax.experimental.pallas import tpu_sc as plsc

N = 10000
E = 160000
D = 256
H = 8
DH = D // H
L = 5
ED = 4
SCALE = math.sqrt(DH)

NP = 10240


def _ln_rows(x, g, b):
    mu = jnp.mean(x, axis=-1, keepdims=True)
    var = jnp.mean((x - mu) ** 2, axis=-1, keepdims=True)
    return (x - mu) * jax.lax.rsqrt(var + 1e-5) * g + b


def _gelu(x):
    return 0.5 * x * (1.0 + lax.erf(x * 0.7071067811865476))


def _dotT(a, w):
    return lax.dot_general(a, w, (((1,), (1,)), ((), ())),
                           preferred_element_type=jnp.float32)


def _in_body(x_ref, w_ref, b_ref, g_ref, bt_ref, o_ref):
    h = _dotT(x_ref[...], w_ref[...]) + b_ref[...]
    o_ref[...] = _gelu(_ln_rows(h, g_ref[...], bt_ref[...]))


def _input_mlp(x, inW, inb, ing, inbt):
    B = 512
    return pl.pallas_call(
        _in_body,
        grid=(NP // B,),
        in_specs=[
            pl.BlockSpec((B, D), lambda i: (i, 0)),
            pl.BlockSpec((D, D), lambda i: (0, 0)),
            pl.BlockSpec((D,), lambda i: (0,)),
            pl.BlockSpec((D,), lambda i: (0,)),
            pl.BlockSpec((D,), lambda i: (0,)),
        ],
        out_specs=pl.BlockSpec((B, D), lambda i: (i, 0)),
        out_shape=jax.ShapeDtypeStruct((NP, D), jnp.float32),
    )(x, inW, inb, ing, inbt)


def _qkv_body(h_ref, g_ref, b_ref, wq_ref, wk_ref, wv_ref, q_ref, k_ref, v_ref):
    hn = _ln_rows(h_ref[...], g_ref[...], b_ref[...])
    q_ref[...] = _dotT(hn, wq_ref[...])
    k_ref[...] = _dotT(hn, wk_ref[...])
    v_ref[...] = _dotT(hn, wv_ref[...])


def _qkv(h, g1, b1, Wq, Wk, Wv):
    B = 512
    out = jax.ShapeDtypeStruct((NP, D), jnp.float32)
    return pl.pallas_call(
        _qkv_body,
        grid=(NP // B,),
        in_specs=[
            pl.BlockSpec((B, D), lambda i: (i, 0)),
            pl.BlockSpec((D,), lambda i: (0,)),
            pl.BlockSpec((D,), lambda i: (0,)),
            pl.BlockSpec((D, D), lambda i: (0, 0)),
            pl.BlockSpec((D, D), lambda i: (0, 0)),
            pl.BlockSpec((D, D), lambda i: (0, 0)),
        ],
        out_specs=[pl.BlockSpec((B, D), lambda i: (i, 0))] * 3,
        out_shape=[out, out, out],
    )(h, g1, b1, Wq, Wk, Wv)


def _post_body(num_ref, den_ref, h_ref, wo_ref, bo_ref, g2_ref, b2_ref,
               f1_ref, bf1_ref, f2_ref, bf2_ref, r_ref, o_ref):
    den_full = jnp.dot(den_ref[...], r_ref[...],
                       preferred_element_type=jnp.float32)
    agg = num_ref[...] / (den_full + 1e-16)
    out = _dotT(agg, wo_ref[...]) + bo_ref[...] + h_ref[...]
    out = _ln_rows(out, g2_ref[...], b2_ref[...])
    f = _gelu(_dotT(out, f1_ref[...]) + bf1_ref[...])
    f = _dotT(f, f2_ref[...]) + bf2_ref[...]
    o_ref[...] = out + f


def _post(num, den, h, Wo, bo, g2, b2, F1, bf1, F2, bf2, R):
    B = 256
    return pl.pallas_call(
        _post_body,
        grid=(NP // B,),
        in_specs=[
            pl.BlockSpec((B, D), lambda i: (i, 0)),
            pl.BlockSpec((B, 16), lambda i: (i, 0)),
            pl.BlockSpec((B, D), lambda i: (i, 0)),
            pl.BlockSpec((D, D), lambda i: (0, 0)),
            pl.BlockSpec((D,), lambda i: (0,)),
            pl.BlockSpec((D,), lambda i: (0,)),
            pl.BlockSpec((D,), lambda i: (0,)),
            pl.BlockSpec((4 * D, D), lambda i: (0, 0)),
            pl.BlockSpec((4 * D,), lambda i: (0,)),
            pl.BlockSpec((D, 4 * D), lambda i: (0, 0)),
            pl.BlockSpec((D,), lambda i: (0,)),
            pl.BlockSpec((16, D), lambda i: (0, 0)),
        ],
        out_specs=pl.BlockSpec((B, D), lambda i: (i, 0)),
        out_shape=jax.ShapeDtypeStruct((NP, D), jnp.float32),
    )(num, den, h, Wo, bo, g2, b2, F1, bf1, F2, bf2, R)


def _cls_body(h_ref, w1_ref, b1_ref, w2_ref, b2_ref, o_ref):
    z = _gelu(_dotT(h_ref[...], w1_ref[...]) + b1_ref[...])
    o_ref[...] = _dotT(z, w2_ref[...]) + b2_ref[...]


def _classifier(h, cW1, cb1, cW2, cb2):
    B = 512
    cW2p = jnp.zeros((128, D // 2), jnp.float32).at[:2].set(cW2)
    cb2p = jnp.zeros((128,), jnp.float32).at[:2].set(cb2)
    out = pl.pallas_call(
        _cls_body,
        grid=(NP // B,),
        in_specs=[
            pl.BlockSpec((B, D), lambda i: (i, 0)),
            pl.BlockSpec((D // 2, D), lambda i: (0, 0)),
            pl.BlockSpec((D // 2,), lambda i: (0,)),
            pl.BlockSpec((128, D // 2), lambda i: (0, 0)),
            pl.BlockSpec((128,), lambda i: (0,)),
        ],
        out_specs=pl.BlockSpec((B, 128), lambda i: (i, 0)),
        out_shape=jax.ShapeDtypeStruct((NP, 128), jnp.float32),
    )(h, cW1, cb1, cW2p, cb2p)
    return out[:N, :2]


def _eb_body(ea_ref, we_ref, be_ref, o_ref):
    w = we_ref[...].reshape(H, ED)
    b = be_ref[...].reshape(1, H)
    o_ref[...] = (_dotT(ea_ref[...], w) + b)[None]


def _edge_bias(ea, We, be):
    BE = 8000
    return pl.pallas_call(
        _eb_body,
        grid=(L, E // BE),
        in_specs=[
            pl.BlockSpec((BE, ED), lambda l, e: (e, 0)),
            pl.BlockSpec((1, H, ED), lambda l, e: (l, 0, 0)),
            pl.BlockSpec((1, 1, H), lambda l, e: (l, 0, 0)),
        ],
        out_specs=pl.BlockSpec((1, BE, H), lambda l, e: (l, e, 0)),
        out_shape=jax.ShapeDtypeStruct((L, E, H), jnp.float32),
    )(ea, We, be.reshape(L, 1, H))


NT = 32
NPT = NP // NT
CE = 32


def _sc_edge(q, k, v, src_s, dst_s, eb_l, offs):
    f32 = jnp.float32
    i32 = jnp.int32
    mesh = plsc.VectorSubcoreMesh(core_axis_name="c", subcore_axis_name="s")

    @functools.partial(
        pl.kernel,
        mesh=mesh,
        compiler_params=pltpu.CompilerParams(needs_layout_passes=False,
                                             use_tc_tiling_on_sc=False),
        out_type=[jax.ShapeDtypeStruct((NP, D), f32),
                  jax.ShapeDtypeStruct((NP, 16), f32)],
        scratch_types=[
            pltpu.VMEM((NPT, D), f32),
            pltpu.VMEM((NPT, 16), f32),
            pltpu.VMEM((CE, D), f32),
            pltpu.VMEM((CE, D), f32),
            pltpu.VMEM((CE, D), f32),
            pltpu.VMEM((CE, H), f32),
            pltpu.VMEM((CE, 16), f32),
            pltpu.VMEM((CE,), i32),
            pltpu.VMEM((CE,), i32),
            pltpu.VMEM((40,), i32),
            pltpu.SemaphoreType.DMA,
            pltpu.SemaphoreType.DMA,
            pltpu.SemaphoreType.DMA,
        ],
    )
    def edge_kernel(q_h, k_h, v_h, src_h, dst_h, eb_h, offs_h,
                    num_h, den_h, acc, dacc, qb, kb, vb, ebb, wb, sb, db,
                    offs_v, sq, sk, sv):
        wid = lax.axis_index("s") * 2 + lax.axis_index("c")
        base = wid * NPT

        def sload(ref, i):
            return plsc.load_gather(ref, [jnp.full((16,), i, i32)])[0]

        pltpu.sync_copy(offs_h, offs_v)
        e_lo = sload(offs_v, wid)
        e_hi = sload(offs_v, wid + 1)

        zero16 = jnp.zeros((16,), f32)
        lanes = jnp.arange(16, dtype=i32)

        def zrow(i, carry):
            for j in range(D // 16):
                acc[i, pl.ds(16 * j, 16)] = zero16
            dacc[i, :] = zero16
            return carry
        lax.fori_loop(0, NPT, zrow, 0)

        def zw(i, carry):
            wb[i, :] = zero16
            return carry
        lax.fori_loop(0, CE, zw, 0)

        start = (e_lo // CE) * CE
        nch = (e_hi - start + CE - 1) // CE

        def chunk(c, carry):
            e0 = start + c * CE
            pltpu.sync_copy(src_h.at[pl.ds(e0, CE)], sb)
            pltpu.sync_copy(dst_h.at[pl.ds(e0, CE)], db)
            pltpu.sync_copy(eb_h.at[pl.ds(e0, CE)], ebb)
            cq = pltpu.async_copy(q_h.at[db], qb, sq)
            ck = pltpu.async_copy(k_h.at[sb], kb, sk)
            cv = pltpu.async_copy(v_h.at[sb], vb, sv)
            cq.wait()
            ck.wait()
            cv.wait()

            for g in range(CE // 16):
                eidx = lanes + (16 * g)
                ge = eidx + e0
                valid = (ge >= e_lo) & (ge < e_hi)
                for h in range(H):
                    def dstep(d, a):
                        dv = jnp.full((16,), DH * h, i32) + d
                        qv = plsc.load_gather(qb, [eidx, dv])
                        kv = plsc.load_gather(kb, [eidx, dv])
                        return a + qv * kv
                    s = lax.fori_loop(0, DH, dstep, zero16)
                    hv = jnp.full((16,), h, i32)
                    ebv = plsc.load_gather(ebb, [eidx, hv])
                    w = jnp.exp(s * (1.0 / SCALE) + ebv)
                    w = jnp.where(valid, w, 0.0)
                    plsc.store_scatter(wb, [eidx, hv], w)

            def edge(ei, carry):
                dl = jnp.clip(sload(db, ei) - base, 0, NPT - 1)
                wrow = wb[ei, :]
                plsc.addupdate(dacc.at[dl], wrow)
                for j in range(D // 16):
                    wh = wrow[j // 2]
                    vv = vb[ei, pl.ds(16 * j, 16)]
                    plsc.addupdate(acc.at[dl, pl.ds(16 * j, 16)], vv * wh)
                return carry
            lax.fori_loop(0, CE, edge, 0)
            return carry

        lax.fori_loop(0, nch, chunk, 0)

        pltpu.sync_copy(acc, num_h.at[pl.ds(base, NPT)])
        pltpu.sync_copy(dacc, den_h.at[pl.ds(base, NPT)])

    return edge_kernel(q, k, v, src_s, dst_s, eb_l, offs)


def kernel(x, edge_index, edge_attr, inW, inb, ing, inbt, Wq, Wk, Wv, We, be,
           Wo, bo, g1, b1, g2, b2, F1, bf1, F2, bf2, cW1, cb1, cW2, cb2):
    src = edge_index[0].astype(jnp.int32)
    dst = edge_index[1].astype(jnp.int32)

    perm = jnp.argsort(dst)
    src_s = src[perm]
    dst_s = dst[perm]
    ea_s = edge_attr[perm]
    offs = jnp.searchsorted(dst_s, jnp.arange(NT + 1, dtype=jnp.int32) * NPT)
    offs = jnp.zeros((40,), jnp.int32).at[:NT + 1].set(offs.astype(jnp.int32))

    xP = jnp.zeros((NP, D), jnp.float32).at[:N].set(x)
    R = jnp.zeros((16, D), jnp.float32)
    R = R.at[jnp.arange(D) // DH, jnp.arange(D)].set(1.0)

    eb_all = _edge_bias(ea_s, We, be)

    h = _input_mlp(xP, inW, inb, ing, inbt)
    for l in range(L):
        q, k, v = _qkv(h, g1[l], b1[l], Wq[l], Wk[l], Wv[l])
        num, den = _sc_edge(q, k, v, src_s, dst_s, eb_all[l], offs)
        h = _post(num, den, h, Wo[l], bo[l], g2[l], b2[l], F1[l], bf1[l],
                  F2[l], bf2[l], R)

    logits = _classifier(h, cW1, cb1, cW2, cb2)
    return logits, h[:N]
